# baseline (speedup 1.0000x reference)
import jax
import jax.numpy as jnp
from jax import lax
from jax.experimental import pallas as pl
from jax.experimental.pallas import tpu as pltpu
from jax.experimental.pallas import tpu_sc as plsc

B, N, D = 8, 2048, 128
NUM_CORES, NUM_SUBCORES = 2, 16
NW = NUM_CORES * NUM_SUBCORES
SUBC_PER_ROW = NW // B
SB_P = 256
SB_F = SB_P * D
NSB = (N // SB_P) // SUBC_PER_ROW
ROW_F = N * D
VEC = 16


def _sc_body(x_hbm, off_hbm, out_hbm, buf, zbuf, offb, sem_off, sem_in, sem_out):
    c = lax.axis_index("c")
    s = lax.axis_index("s")
    wid = c * NUM_SUBCORES + s
    b = wid % B
    q = wid // B
    row_base = b * ROW_F

    def sb_pos(k):
        return (q + SUBC_PER_ROW * k) * SB_P

    for k in range(NSB):
        pltpu.async_copy(
            x_hbm.at[pl.ds(row_base + sb_pos(k) * D, SB_F)],
            buf.at[pl.ds(k * SB_F, SB_F)],
            sem_in.at[k],
        )

    off_copy = pltpu.make_async_copy(off_hbm, offb.at[pl.ds(0, B + 1)], sem_off)
    off_copy.start()

    zero = jnp.zeros((VEC,), jnp.float32)

    def zfill(p, carry):
        for u in range(D // VEC):
            zbuf[pl.ds(p * D + u * VEC, VEC)] = zero
        return carry

    lax.fori_loop(0, SB_P, zfill, 0)

    off_copy.wait()
    offv = offb[pl.ds(b, VEC)]
    nv = jnp.clip(offv[1] - offv[0], 0, N)

    for k in range(NSB):
        @pl.when(sb_pos(k) >= nv)
        def _(k=k):
            pltpu.async_copy(
                zbuf, out_hbm.at[pl.ds(row_base + sb_pos(k) * D, SB_F)], sem_out
            )

    for k in range(NSB):
        @pl.when(sb_pos(k) < nv)
        def _(k=k):
            pltpu.make_async_copy(
                x_hbm.at[pl.ds(row_base + sb_pos(k) * D, SB_F)],
                buf.at[pl.ds(k * SB_F, SB_F)],
                sem_in.at[k],
            ).wait()

            nvk = jnp.minimum(nv - sb_pos(k), SB_P)

            def ztail(p, carry):
                for u in range(D // VEC):
                    buf[pl.ds(k * SB_F + p * D + u * VEC, VEC)] = zero
                return carry

            lax.fori_loop(nvk, SB_P, ztail, 0)

            pltpu.async_copy(
                buf.at[pl.ds(k * SB_F, SB_F)],
                out_hbm.at[pl.ds(row_base + sb_pos(k) * D, SB_F)],
                sem_out,
            )

    for k in range(NSB):
        pltpu.make_async_copy(
            zbuf, out_hbm.at[pl.ds(0, SB_F)], sem_out
        ).wait()

        @pl.when(sb_pos(k) >= nv)
        def _(k=k):
            pltpu.make_async_copy(
                x_hbm.at[pl.ds(row_base + sb_pos(k) * D, SB_F)],
                buf.at[pl.ds(k * SB_F, SB_F)],
                sem_in.at[k],
            ).wait()


def kernel(x, x_offsets, all_timestamps, invalid_attn_mask):
    del all_timestamps, invalid_attn_mask
    xf = x.reshape(-1)
    off = x_offsets.astype(jnp.int32)
    mesh = plsc.VectorSubcoreMesh(core_axis_name="c", subcore_axis_name="s")
    fn = pl.kernel(
        _sc_body,
        mesh=mesh,
        out_type=jax.ShapeDtypeStruct((B * N * D,), jnp.float32),
        scratch_types=[
            pltpu.VMEM((NSB * SB_F,), jnp.float32),
            pltpu.VMEM((SB_F,), jnp.float32),
            pltpu.VMEM((32,), jnp.int32),
            pltpu.SemaphoreType.DMA,
            pltpu.SemaphoreType.DMA((NSB,)),
            pltpu.SemaphoreType.DMA,
        ],
    )
    return fn(xf, off).reshape(B, N, D)


# speedup vs baseline: 1.0199x; 1.0199x over previous
import jax
import jax.numpy as jnp
from jax import lax
from jax.experimental import pallas as pl
from jax.experimental.pallas import tpu as pltpu
from jax.experimental.pallas import tpu_sc as plsc

B, N, D = 8, 2048, 128
NUM_CORES, NUM_SUBCORES = 2, 16
NW = NUM_CORES * NUM_SUBCORES
SUBC_PER_ROW = NW // B
SB_P = 128
NSB = (N // SB_P) // SUBC_PER_ROW
ROW_F = N * D
VEC = 16
ZB_P = 64
ZB_F = ZB_P * D

ENTRIES = [(0, 0, 32), (0, 32, 32), (0, 64, 64),
           (1, 0, 128), (2, 0, 128), (3, 0, 128)]
NE = len(ENTRIES)
TOT_P = SB_P * NSB


def _sc_body(x_hbm, off_hbm, out_hbm, buf, zbuf, offb, sem_off, sem_in, sem_out):
    c = lax.axis_index("c")
    s = lax.axis_index("s")
    wid = c * NUM_SUBCORES + s
    b = wid % B
    q = wid // B
    row_base = b * ROW_F

    def entry_refs(e):
        k, off, size = ENTRIES[e]
        pos = (q + SUBC_PER_ROW * k) * SB_P + off
        hslice = pl.ds(row_base + pos * D, size * D)
        bslice = pl.ds((k * SB_P + off) * D, size * D)
        return pos, size, x_hbm.at[hslice], out_hbm.at[hslice], buf.at[bslice]

    for e in range(NE):
        _, _, src, _, stage = entry_refs(e)
        pltpu.async_copy(src, stage, sem_in.at[e])

    off_copy = pltpu.make_async_copy(off_hbm, offb.at[pl.ds(0, B + 1)], sem_off)
    off_copy.start()

    zero = jnp.zeros((VEC,), jnp.float32)

    def zfill(p, carry):
        for u in range(D // VEC):
            zbuf[pl.ds(p * D + u * VEC, VEC)] = zero
        return carry

    lax.fori_loop(0, ZB_P, zfill, 0)

    off_copy.wait()
    offv = offb[pl.ds(b, VEC)]
    nv = jnp.clip(offv[1] - offv[0], 0, N)

    for e in range(NE):
        pos, size, _, dst, _ = entry_refs(e)

        @pl.when(pos >= nv)
        def _(pos=pos, size=size):
            base = row_base + pos * D
            for piece in range(0, size, ZB_P):
                psize = min(ZB_P, size - piece)
                pltpu.async_copy(
                    zbuf.at[pl.ds(0, psize * D)],
                    out_hbm.at[pl.ds(base + piece * D, psize * D)],
                    sem_out,
                )

    for e in range(NE):
        pos, size, src, dst, stage = entry_refs(e)

        @pl.when(pos < nv)
        def _(pos=pos, size=size, src=src, dst=dst, stage=stage, e=e):
            pltpu.make_async_copy(src, stage, sem_in.at[e]).wait()

            nve = jnp.minimum(nv - pos, size)
            k, off, _ = ENTRIES[e]
            ebase = (k * SB_P + off) * D

            def ztail(p, carry):
                for u in range(D // VEC):
                    buf[pl.ds(ebase + p * D + u * VEC, VEC)] = zero
                return carry

            lax.fori_loop(nve, size, ztail, 0)
            pltpu.async_copy(stage, dst, sem_out)

    for _ in range(TOT_P // ZB_P):
        pltpu.make_async_copy(zbuf, out_hbm.at[pl.ds(0, ZB_F)], sem_out).wait()

    for e in range(NE):
        pos, _, src, _, stage = entry_refs(e)

        @pl.when(pos >= nv)
        def _(src=src, stage=stage, e=e):
            pltpu.make_async_copy(src, stage, sem_in.at[e]).wait()


def kernel(x, x_offsets, all_timestamps, invalid_attn_mask):
    del all_timestamps, invalid_attn_mask
    xf = x.reshape(-1)
    off = x_offsets.astype(jnp.int32)
    mesh = plsc.VectorSubcoreMesh(core_axis_name="c", subcore_axis_name="s")
    fn = pl.kernel(
        _sc_body,
        mesh=mesh,
        out_type=jax.ShapeDtypeStruct((B * N * D,), jnp.float32),
        scratch_types=[
            pltpu.VMEM((NSB * SB_P * D,), jnp.float32),
            pltpu.VMEM((ZB_F,), jnp.float32),
            pltpu.VMEM((32,), jnp.int32),
            pltpu.SemaphoreType.DMA,
            pltpu.SemaphoreType.DMA((NE,)),
            pltpu.SemaphoreType.DMA,
        ],
    )
    return fn(xf, off).reshape(B, N, D)


# speedup vs baseline: 1.0325x; 1.0124x over previous
import jax
import jax.numpy as jnp
from jax import lax
from jax.experimental import pallas as pl
from jax.experimental.pallas import tpu as pltpu
from jax.experimental.pallas import tpu_sc as plsc

B, N, D = 8, 2048, 128
NUM_CORES, NUM_SUBCORES = 2, 16
NW = NUM_CORES * NUM_SUBCORES
SUBC_PER_ROW = NW // B
SB_P = 128
SB_F = SB_P * D
NSB = (N // SB_P) // SUBC_PER_ROW
ROW_F = N * D
VEC = 16


def _sc_body(x_hbm, off_hbm, out_hbm, buf, zbuf, offb, sem_off, sem_in, sem_out):
    c = lax.axis_index("c")
    s = lax.axis_index("s")
    wid = c * NUM_SUBCORES + s
    b = wid % B
    q = wid // B
    row_base = b * ROW_F

    def sb_pos(k):
        return (q + SUBC_PER_ROW * k) * SB_P

    for k in range(NSB):
        pltpu.async_copy(
            x_hbm.at[pl.ds(row_base + sb_pos(k) * D, SB_F)],
            buf.at[pl.ds(k * SB_F, SB_F)],
            sem_in.at[k],
        )

    off_copy = pltpu.make_async_copy(off_hbm, offb.at[pl.ds(0, B + 1)], sem_off)
    off_copy.start()

    zero = jnp.zeros((VEC,), jnp.float32)

    def zfill(p, carry):
        for u in range(D // VEC):
            zbuf[pl.ds(p * D + u * VEC, VEC)] = zero
        return carry

    lax.fori_loop(0, SB_P, zfill, 0)

    off_copy.wait()
    offv = offb[pl.ds(b, VEC)]
    nv = jnp.clip(offv[1] - offv[0], 0, N)

    for k in range(NSB):
        @pl.when(sb_pos(k) >= nv)
        def _(k=k):
            pltpu.async_copy(
                zbuf, out_hbm.at[pl.ds(row_base + sb_pos(k) * D, SB_F)], sem_out
            )

    for k in range(NSB):
        @pl.when(sb_pos(k) < nv)
        def _(k=k):
            pltpu.make_async_copy(
                x_hbm.at[pl.ds(row_base + sb_pos(k) * D, SB_F)],
                buf.at[pl.ds(k * SB_F, SB_F)],
                sem_in.at[k],
            ).wait()

            nvk = jnp.minimum(nv - sb_pos(k), SB_P)

            def ztail(p, carry):
                for u in range(D // VEC):
                    buf[pl.ds(k * SB_F + p * D + u * VEC, VEC)] = zero
                return carry

            lax.fori_loop(nvk, SB_P, ztail, 0)

            pltpu.async_copy(
                buf.at[pl.ds(k * SB_F, SB_F)],
                out_hbm.at[pl.ds(row_base + sb_pos(k) * D, SB_F)],
                sem_out,
            )

    for k in range(NSB):
        pltpu.make_async_copy(
            zbuf, out_hbm.at[pl.ds(0, SB_F)], sem_out
        ).wait()

        @pl.when(sb_pos(k) >= nv)
        def _(k=k):
            pltpu.make_async_copy(
                x_hbm.at[pl.ds(row_base + sb_pos(k) * D, SB_F)],
                buf.at[pl.ds(k * SB_F, SB_F)],
                sem_in.at[k],
            ).wait()


def kernel(x, x_offsets, all_timestamps, invalid_attn_mask):
    del all_timestamps, invalid_attn_mask
    xf = x.reshape(-1)
    off = x_offsets.astype(jnp.int32)
    mesh = plsc.VectorSubcoreMesh(core_axis_name="c", subcore_axis_name="s")
    fn = pl.kernel(
        _sc_body,
        mesh=mesh,
        out_type=jax.ShapeDtypeStruct((B * N * D,), jnp.float32),
        scratch_types=[
            pltpu.VMEM((NSB * SB_F,), jnp.float32),
            pltpu.VMEM((SB_F,), jnp.float32),
            pltpu.VMEM((32,), jnp.int32),
            pltpu.SemaphoreType.DMA,
            pltpu.SemaphoreType.DMA((NSB,)),
            pltpu.SemaphoreType.DMA,
        ],
    )
    return fn(xf, off).reshape(B, N, D)


# speedup vs baseline: 1.0355x; 1.0029x over previous
import jax
import jax.numpy as jnp
from jax import lax
from jax.experimental import pallas as pl
from jax.experimental.pallas import tpu as pltpu
from jax.experimental.pallas import tpu_sc as plsc

B, N, D = 8, 2048, 128
NUM_CORES, NUM_SUBCORES = 2, 16
NW = NUM_CORES * NUM_SUBCORES
SUBC_PER_ROW = NW // B
SB_P = 128
SB_F = SB_P * D
NSB = (N // SB_P) // SUBC_PER_ROW
ROW_F = N * D
VEC = 16
ZB_P = 64
ZB_F = ZB_P * D


def _sc_body(x_hbm, off_hbm, out_hbm, buf, zbuf, offb, sem_off, sem_in, sem_out):
    c = lax.axis_index("c")
    s = lax.axis_index("s")
    wid = c * NUM_SUBCORES + s
    b = wid % B
    q = wid // B
    row_base = b * ROW_F

    def sb_pos(k):
        return (q + SUBC_PER_ROW * k) * SB_P

    for k in range(NSB):
        pltpu.async_copy(
            x_hbm.at[pl.ds(row_base + sb_pos(k) * D, SB_F)],
            buf.at[pl.ds(k * SB_F, SB_F)],
            sem_in.at[k],
        )

    off_copy = pltpu.make_async_copy(off_hbm, offb.at[pl.ds(0, B + 1)], sem_off)
    off_copy.start()

    zero = jnp.zeros((VEC,), jnp.float32)

    def zfill(p, carry):
        for u in range(D // VEC):
            zbuf[pl.ds(p * D + u * VEC, VEC)] = zero
        return carry

    lax.fori_loop(0, ZB_P, zfill, 0)

    off_copy.wait()
    offv = offb[pl.ds(b, VEC)]
    nv = jnp.clip(offv[1] - offv[0], 0, N)

    for k in range(NSB):
        @pl.when(sb_pos(k) >= nv)
        def _(k=k):
            for piece in range(SB_P // ZB_P):
                pltpu.async_copy(
                    zbuf,
                    out_hbm.at[
                        pl.ds(row_base + (sb_pos(k) + piece * ZB_P) * D, ZB_F)
                    ],
                    sem_out,
                )

    for k in range(NSB):
        @pl.when(sb_pos(k) < nv)
        def _(k=k):
            pltpu.make_async_copy(
                x_hbm.at[pl.ds(row_base + sb_pos(k) * D, SB_F)],
                buf.at[pl.ds(k * SB_F, SB_F)],
                sem_in.at[k],
            ).wait()

            nvk = jnp.minimum(nv - sb_pos(k), SB_P)

            def ztail(p, carry):
                for u in range(D // VEC):
                    buf[pl.ds(k * SB_F + p * D + u * VEC, VEC)] = zero
                return carry

            lax.fori_loop(nvk, SB_P, ztail, 0)

            pltpu.async_copy(
                buf.at[pl.ds(k * SB_F, SB_F)],
                out_hbm.at[pl.ds(row_base + sb_pos(k) * D, SB_F)],
                sem_out,
            )

    for k in range(NSB):
        pltpu.make_async_copy(
            x_hbm.at[pl.ds(0, SB_F)], buf.at[pl.ds(0, SB_F)], sem_out
        ).wait()

        @pl.when(sb_pos(k) >= nv)
        def _(k=k):
            pltpu.make_async_copy(
                x_hbm.at[pl.ds(row_base + sb_pos(k) * D, SB_F)],
                buf.at[pl.ds(k * SB_F, SB_F)],
                sem_in.at[k],
            ).wait()


def kernel(x, x_offsets, all_timestamps, invalid_attn_mask):
    del all_timestamps, invalid_attn_mask
    xf = x.reshape(-1)
    off = x_offsets.astype(jnp.int32)
    mesh = plsc.VectorSubcoreMesh(core_axis_name="c", subcore_axis_name="s")
    fn = pl.kernel(
        _sc_body,
        mesh=mesh,
        out_type=jax.ShapeDtypeStruct((B * N * D,), jnp.float32),
        scratch_types=[
            pltpu.VMEM((NSB * SB_F,), jnp.float32),
            pltpu.VMEM((ZB_F,), jnp.float32),
            pltpu.VMEM((32,), jnp.int32),
            pltpu.SemaphoreType.DMA,
            pltpu.SemaphoreType.DMA((NSB,)),
            pltpu.SemaphoreType.DMA,
        ],
    )
    return fn(xf, off).reshape(B, N, D)
